# baseline (speedup 1.0000x reference)
import jax
import jax.numpy as jnp
from jax import lax
from jax.experimental import pallas as pl
from jax.experimental.pallas import tpu as pltpu
from jax.experimental.pallas import tpu_sc as plsc

_N = 32 * 1000
_NCLS = 91
_NW = 32
_PER = 1024
_PAD = _NW * _PER - _N
_LIMIT = _N * _NCLS - 1
_CHUNK = 128
_NCHUNK = _PER // _CHUNK


def _sc_body(cls_hbm, tb_hbm, pred_hbm, out_hbm, cls_v, tb_v, idx_v, sub_v,
             buf0, buf1, acc_v, sem0, sem1):
    wid = lax.axis_index("s") * 2 + lax.axis_index("c")
    base = wid * _PER

    pltpu.sync_copy(cls_hbm.at[pl.ds(base * 1, _PER)], cls_v)
    pltpu.sync_copy(tb_hbm.at[pl.ds(base * 4, _PER * 4)], tb_v)

    iota = lax.iota(jnp.int32, 16)
    ii4 = iota >> 2
    im4 = iota & 3

    bufs = (buf0, buf1)
    sems = (sem0, sem1)
    copies = [None] * _NCHUNK

    def fire(c):
        copies[c] = pltpu.async_copy(pred_hbm.at[idx_v.at[c]], bufs[c % 2],
                                     sems[c % 2])

    for c in range(_NCHUNK):
        for k in range(_CHUNK // 16):
            i = c * _CHUNK + k * 16
            cls16 = cls_v[pl.ds(i, 16)]
            q = jnp.minimum((base + i + iota) * _NCLS + cls16, _LIMIT)
            idx_v[c, pl.ds(k * 16, 16)] = q >> 5
            sub_v[pl.ds(i, 16)] = (q & 31) * 4
        if c < 2:
            fire(c)

    def chunk_body(c, buf):
        def body(j, carry):
            acc, cnt = carry
            e = c * _CHUNK * 4 + j * 16
            t = tb_v[pl.ds(e, 16)]
            rloc = j * 4 + ii4
            r4 = c * _CHUNK + rloc
            off = plsc.load_gather(sub_v, [r4]) + im4
            p = plsc.load_gather(buf, [rloc, off])
            c16 = plsc.load_gather(cls_v, [r4])
            d = jnp.abs(t - p)
            l = jnp.where(d < 1.0, 0.5 * d * d, d - 0.5)
            m = c16 > 0
            return acc + jnp.where(m, l, 0.0), cnt + jnp.where(m, 1.0, 0.0)
        return body

    zero = jnp.zeros((16,), jnp.float32)
    carry = (zero, zero)
    for c in range(_NCHUNK):
        copies[c].wait()
        carry = lax.fori_loop(0, (_CHUNK * 4) // 16, chunk_body(c, bufs[c % 2]),
                              carry)
        if c + 2 < _NCHUNK:
            fire(c + 2)
    acc, cnt = carry
    acc_v[pl.ds(0, 16)] = acc
    acc_v[pl.ds(16, 16)] = cnt
    pltpu.sync_copy(acc_v, out_hbm.at[wid])


def _tc_finish(part_ref, out_ref):
    p = part_ref[...]
    total = jnp.sum(p[:, :16])
    count = jnp.sum(p[:, 16:])
    loss = jnp.where(count > 0, total / jnp.maximum(count, 1.0), 0.0)
    out_ref[...] = jnp.reshape(loss, (1, 1))


def kernel(target_boxes, target_class_ids, pred_boxes):
    cls = target_class_ids.reshape(-1).astype(jnp.int32)
    cls = jnp.pad(cls, (0, _PAD))
    tb = jnp.pad(target_boxes.reshape(-1, 4), ((0, _PAD), (0, 0))).reshape(-1)
    pred = pred_boxes.reshape(-1, 128)

    mesh = plsc.VectorSubcoreMesh(core_axis_name="c", subcore_axis_name="s")
    sc = pl.kernel(
        _sc_body, mesh=mesh,
        compiler_params=pltpu.CompilerParams(needs_layout_passes=False),
        out_type=jax.ShapeDtypeStruct((_NW, 32), jnp.float32),
        scratch_types=[
            pltpu.VMEM((_PER,), jnp.int32),
            pltpu.VMEM((_PER * 4,), jnp.float32),
            pltpu.VMEM((_NCHUNK, _CHUNK), jnp.int32),
            pltpu.VMEM((_PER,), jnp.int32),
            pltpu.VMEM((_CHUNK, 128), jnp.float32),
            pltpu.VMEM((_CHUNK, 128), jnp.float32),
            pltpu.VMEM((32,), jnp.float32),
            pltpu.SemaphoreType.DMA,
            pltpu.SemaphoreType.DMA,
        ],
    )
    partials = sc(cls, tb, pred)

    out = pl.pallas_call(
        _tc_finish,
        out_shape=jax.ShapeDtypeStruct((1, 1), jnp.float32),
    )(partials)
    return out[0, 0]


# speedup vs baseline: 27.8741x; 27.8741x over previous
import jax
import jax.numpy as jnp
from jax.experimental import pallas as pl
from jax.experimental.pallas import tpu as pltpu

_B = 32
_NCLS = 91
_R = 1000
_CC = 7
_CB = 13


def _body(cls_ref, tb_ref, pred_ref, out_ref, acc):
    b = pl.program_id(0)
    cc = pl.program_id(1)

    @pl.when(jnp.logical_and(b == 0, cc == 0))
    def _init():
        acc[0] = 0.0
        acc[1] = 0.0

    tb = tb_ref[0]
    cls_row = cls_ref[0]
    part = jnp.zeros((4, _R), jnp.float32)
    for j in range(_CB):
        c = None
        cid = cc * _CB + j
        p = pred_ref[0, j]
        d = jnp.abs(tb - p)
        l = jnp.where(d < 1.0, 0.5 * d * d, d - 0.5)
        m = jnp.logical_and(cls_row == cid, cls_row > 0)
        part = part + jnp.where(jnp.broadcast_to(m, (4, _R)), l, 0.0)
    acc[0] = acc[0] + jnp.sum(part)

    @pl.when(cc == 0)
    def _count():
        acc[1] = acc[1] + 4.0 * jnp.sum((cls_row > 0).astype(jnp.float32))

    @pl.when(jnp.logical_and(b == _B - 1, cc == _CC - 1))
    def _fin():
        total, count = acc[0], acc[1]
        out_ref[...] = jnp.reshape(
            jnp.where(count > 0, total / jnp.maximum(count, 1.0), 0.0), (1, 1))


def kernel(target_boxes, target_class_ids, pred_boxes):
    cls = target_class_ids.astype(jnp.int32).reshape(_B, 1, _R)
    tb = target_boxes.transpose(0, 2, 1)
    pred = pred_boxes.transpose(0, 2, 3, 1)

    out = pl.pallas_call(
        _body,
        grid=(_B, _CC),
        in_specs=[
            pl.BlockSpec((1, 1, _R), lambda b, cc: (b, 0, 0)),
            pl.BlockSpec((1, 4, _R), lambda b, cc: (b, 0, 0)),
            pl.BlockSpec((1, _CB, 4, _R), lambda b, cc: (b, cc, 0, 0)),
        ],
        out_specs=pl.BlockSpec((1, 1), lambda b, cc: (0, 0)),
        out_shape=jax.ShapeDtypeStruct((1, 1), jnp.float32),
        scratch_shapes=[pltpu.SMEM((2,), jnp.float32)],
    )(cls, tb, pred)
    return out[0, 0]


# speedup vs baseline: 108.5914x; 3.8958x over previous
import jax
import jax.numpy as jnp
from jax.experimental import pallas as pl
from jax.experimental.pallas import tpu as pltpu

_B = 32
_NCLS = 91
_R = 1000


def _body(cls_ref, tb_ref, pred_ref, out_ref, acc):
    b = pl.program_id(0)

    @pl.when(b == 0)
    def _init():
        acc[0] = 0.0
        acc[1] = 0.0

    cls_row = cls_ref[0]
    tb = tb_ref[0]

    psel = jnp.zeros((4, _R), jnp.float32)
    for c in range(1, _NCLS):
        psel = psel + jnp.where(cls_row == c, pred_ref[0, c], 0.0)

    d = jnp.abs(tb - psel)
    l = jnp.where(d < 1.0, 0.5 * d * d, d - 0.5)
    valid = cls_row > 0
    acc[0] = acc[0] + jnp.sum(jnp.where(valid, l, 0.0))
    acc[1] = acc[1] + 4.0 * jnp.sum(valid.astype(jnp.float32))

    @pl.when(b == _B - 1)
    def _fin():
        total, count = acc[0], acc[1]
        out_ref[...] = jnp.reshape(
            jnp.where(count > 0, total / jnp.maximum(count, 1.0), 0.0), (1, 1))


def kernel(target_boxes, target_class_ids, pred_boxes):
    cls = target_class_ids.astype(jnp.int32).reshape(_B, 1, _R)
    tb = target_boxes.transpose(0, 2, 1)
    pred = pred_boxes.transpose(0, 2, 3, 1)

    out = pl.pallas_call(
        _body,
        grid=(_B,),
        in_specs=[
            pl.BlockSpec((1, 1, _R), lambda b: (b, 0, 0)),
            pl.BlockSpec((1, 4, _R), lambda b: (b, 0, 0)),
            pl.BlockSpec((1, _NCLS, 4, _R), lambda b: (b, 0, 0, 0)),
        ],
        out_specs=pl.BlockSpec((1, 1), lambda b: (0, 0)),
        out_shape=jax.ShapeDtypeStruct((1, 1), jnp.float32),
        scratch_shapes=[pltpu.SMEM((2,), jnp.float32)],
    )(cls, tb, pred)
    return out[0, 0]
